# Initial kernel scaffold; baseline (speedup 1.0000x reference)
import jax
import jax.numpy as jnp
from jax.experimental import pallas as pl


def kernel(pc, batch_idx):
    raise NotImplementedError("write your pallas kernel here")


# speedup vs baseline: 52.7821x; 52.7821x over previous
import functools

import numpy as np
import jax
import jax.numpy as jnp
from jax import lax
from jax.experimental import pallas as pl
from jax.experimental.pallas import tpu as pltpu
from jax.experimental.pallas import tpu_sc as plsc

_SPATIAL = (128, 128, 16)
_LO = np.array([0.0, -25.6, -2.0], dtype=np.float32)
_HI = np.array([51.2, 25.6, 4.4], dtype=np.float32)
_RANGE = _HI - _LO
_TLO = (_LO.astype(np.float64) + 1e-4).astype(np.float32)
_THI = (_HI.astype(np.float64) - 1e-4).astype(np.float32)

_N = 529408
_NC, _NS, _L = 2, 16, 16
_PT = _N // _NS
_CHUNK = 4096
_NFULL = _PT // _CHUNK
_REM = _PT - _NFULL * _CHUNK
_NCHUNKS = _NFULL + 1
_CELLS = _SPATIAL[0] * _SPATIAL[1] * _SPATIAL[2]
_HALF = _CELLS // 2
_PAD = 1024
_SLAB = _HALF // _NS


def _make_kernel():
    mesh = plsc.VectorSubcoreMesh(core_axis_name="c", subcore_axis_name="s")

    @functools.partial(
        pl.kernel,
        out_type=jax.ShapeDtypeStruct((_CELLS,), jnp.float32),
        mesh=mesh,
        compiler_params=pltpu.CompilerParams(needs_layout_passes=False),
        scratch_types=[
            pltpu.VMEM((_CHUNK,), jnp.float32),
            pltpu.VMEM((_CHUNK,), jnp.float32),
            pltpu.VMEM((_CHUNK,), jnp.float32),
            pltpu.VMEM((_CHUNK,), jnp.float32),
            pltpu.VMEM((_CHUNK,), jnp.float32),
            pltpu.VMEM((_CHUNK,), jnp.float32),
            pltpu.VMEM((_CHUNK,), jnp.int32),
            pltpu.VMEM((_CHUNK,), jnp.int32),
            pltpu.VMEM((_CHUNK,), jnp.float32),
            pltpu.VMEM((_SLAB,), jnp.float32),
            pltpu.VMEM_SHARED((_HALF + _PAD,), jnp.float32),
            pltpu.SemaphoreType.DMA,
            pltpu.SemaphoreType.DMA,
            pltpu.SemaphoreType.DMA,
            pltpu.SemaphoreType.DMA,
        ],
    )
    def vox(x_hbm, y_hbm, z_hbm, out_hbm,
            xv0, xv1, yv0, yv1, zv0, zv1, kb0, kb1, ones_v, stage_v, grid_sh,
            lsem0, lsem1, ssem0, ssem1):
        c = lax.axis_index("c")
        s = lax.axis_index("s")
        iota = lax.iota(jnp.int32, _L)
        zeros16 = jnp.zeros((_L,), jnp.float32)
        ones16 = jnp.ones((_L,), jnp.float32)

        kbufs = (kb0, kb1)
        lsems = (lsem0, lsem1)
        ssems = (ssem0, ssem1)
        xbufs = (xv0, xv1)
        ybufs = (yv0, yv1)
        zbufs = (zv0, zv1)
        pts_base = s * _PT

        def issue_load(k):
            b = k % 2
            cnt = _CHUNK if k < _NFULL else _REM
            base = pl.multiple_of(pts_base + k * _CHUNK, 8)
            cps = []
            for hbm, buf in ((x_hbm, xbufs[b]), (y_hbm, ybufs[b]),
                             (z_hbm, zbufs[b])):
                dst = buf if cnt == _CHUNK else buf.at[pl.ds(0, cnt)]
                cps.append(pltpu.async_copy(
                    hbm.at[pl.ds(base, cnt)], dst, lsems[b]))
            return cps

        loads = {0: issue_load(0), 1: issue_load(1)}

        def fill_stage(i, carry):
            stage_v[pl.ds(i * _L, _L)] = zeros16
            return carry

        lax.fori_loop(0, _SLAB // _L, fill_stage, 0)

        def fill_ones(i, carry):
            ones_v[pl.ds(i * _L, _L)] = ones16
            return carry

        lax.fori_loop(0, _CHUNK // _L, fill_ones, 0)

        pltpu.sync_copy(stage_v, grid_sh.at[pl.ds(s * _SLAB, _SLAB)])
        plsc.subcore_barrier()

        half_lo = c * _HALF

        def make_point_body(b):
            def point_body(i, carry):
                o = i * _L
                x = xbufs[b][pl.ds(o, _L)]
                y = ybufs[b][pl.ds(o, _L)]
                z = zbufs[b][pl.ds(o, _L)]
                m = (
                    (x > _TLO[0]) & (x < _THI[0])
                    & (y > _TLO[1]) & (y < _THI[1])
                    & (z > _TLO[2]) & (z < _THI[2])
                )
                xq = ((x - _LO[0]) * float(_SPATIAL[0]) / _RANGE[0]).astype(jnp.int32)
                yq = ((y - _LO[1]) * float(_SPATIAL[1]) / _RANGE[1]).astype(jnp.int32)
                zq = ((z - _LO[2]) * float(_SPATIAL[2]) / _RANGE[2]).astype(jnp.int32)
                key = (xq * _SPATIAL[1] + yq) * _SPATIAL[2] + zq
                valid = m & (lax.shift_right_logical(key, 17) == c)
                dump = _HALF + (i & 63) * _L + iota
                kbufs[b][pl.ds(o, _L)] = jnp.where(valid, key - half_lo, dump)
                return carry
            return point_body

        scats = {}
        for k in range(_NCHUNKS):
            b = k % 2
            for cp in loads.pop(k):
                cp.wait()
            if k >= 2:
                scats.pop(k - 2).wait()
            iters = (_CHUNK if k < _NFULL else _REM) // _L
            lax.fori_loop(0, iters, make_point_body(b), 0)
            scats[k] = pltpu.async_copy(
                ones_v, grid_sh.at[kbufs[b]], ssems[b])
            if k + 2 < _NCHUNKS:
                loads[k + 2] = issue_load(k + 2)

        for k in sorted(scats):
            scats.pop(k).wait()

        plsc.subcore_barrier()

        pltpu.sync_copy(grid_sh.at[pl.ds(s * _SLAB, _SLAB)], stage_v)
        out_base = pl.multiple_of(c * _HALF + s * _SLAB, 8)
        pltpu.sync_copy(stage_v, out_hbm.at[pl.ds(out_base, _SLAB)])

    return vox


_VOX = _make_kernel()


def kernel(pc, batch_idx):
    del batch_idx
    flat = _VOX(pc[:, 0], pc[:, 1], pc[:, 2])
    return flat.reshape(_SPATIAL)


# speedup vs baseline: 62.9607x; 1.1928x over previous
import functools

import numpy as np
import jax
import jax.numpy as jnp
from jax import lax
from jax.experimental import pallas as pl
from jax.experimental.pallas import tpu as pltpu
from jax.experimental.pallas import tpu_sc as plsc

_SPATIAL = (128, 128, 16)
_LO = np.array([0.0, -25.6, -2.0], dtype=np.float32)
_HI = np.array([51.2, 25.6, 4.4], dtype=np.float32)
_RANGE = _HI - _LO
_TLO = (_LO.astype(np.float64) + 1e-4).astype(np.float32)
_THI = (_HI.astype(np.float64) - 1e-4).astype(np.float32)

_N = 529408
_NC, _NS, _L = 2, 16, 16
_PT = _N // _NS
_CHUNK = 4096
_NFULL = _PT // _CHUNK
_REM = _PT - _NFULL * _CHUNK
_NCHUNKS = _NFULL + 1
_CELLS = _SPATIAL[0] * _SPATIAL[1] * _SPATIAL[2]
_HALF = _CELLS // 2
_PAD = 1024
_SLAB = _HALF // _NS


def _make_kernel():
    mesh = plsc.VectorSubcoreMesh(core_axis_name="c", subcore_axis_name="s")

    @functools.partial(
        pl.kernel,
        out_type=jax.ShapeDtypeStruct((_CELLS,), jnp.float32),
        mesh=mesh,
        compiler_params=pltpu.CompilerParams(needs_layout_passes=False),
        scratch_types=[
            pltpu.VMEM((_CHUNK,), jnp.float32),
            pltpu.VMEM((_CHUNK,), jnp.float32),
            pltpu.VMEM((_CHUNK,), jnp.float32),
            pltpu.VMEM((_CHUNK,), jnp.float32),
            pltpu.VMEM((_CHUNK,), jnp.float32),
            pltpu.VMEM((_CHUNK,), jnp.float32),
            pltpu.VMEM((_CHUNK,), jnp.int32),
            pltpu.VMEM((_CHUNK,), jnp.int32),
            pltpu.VMEM((_CHUNK,), jnp.float32),
            pltpu.VMEM((_SLAB,), jnp.float32),
            pltpu.VMEM_SHARED((_HALF + _PAD,), jnp.float32),
            pltpu.SemaphoreType.DMA,
            pltpu.SemaphoreType.DMA,
            pltpu.SemaphoreType.DMA,
            pltpu.SemaphoreType.DMA,
        ],
    )
    def vox(x_hbm, y_hbm, z_hbm, out_hbm,
            xv0, xv1, yv0, yv1, zv0, zv1, kb0, kb1, ones_v, stage_v, grid_sh,
            lsem0, lsem1, ssem0, ssem1):
        c = lax.axis_index("c")
        s = lax.axis_index("s")
        iota = lax.iota(jnp.int32, _L)
        zeros16 = jnp.zeros((_L,), jnp.float32)
        ones16 = jnp.ones((_L,), jnp.float32)

        kbufs = (kb0, kb1)
        lsems = (lsem0, lsem1)
        ssems = (ssem0, ssem1)
        xbufs = (xv0, xv1)
        ybufs = (yv0, yv1)
        zbufs = (zv0, zv1)
        pts_base = s * _PT

        def issue_load(k):
            b = k % 2
            cnt = _CHUNK if k < _NFULL else _REM
            base = pl.multiple_of(pts_base + k * _CHUNK, 8)
            cps = []
            for hbm, buf in ((x_hbm, xbufs[b]), (y_hbm, ybufs[b]),
                             (z_hbm, zbufs[b])):
                dst = buf if cnt == _CHUNK else buf.at[pl.ds(0, cnt)]
                cps.append(pltpu.async_copy(
                    hbm.at[pl.ds(base, cnt)], dst, lsems[b]))
            return cps

        loads = {0: issue_load(0), 1: issue_load(1)}

        def fill_stage(i, carry):
            stage_v[pl.ds(i * _L, _L)] = zeros16
            return carry

        lax.fori_loop(0, _SLAB // _L, fill_stage, 0)

        def fill_ones(i, carry):
            ones_v[pl.ds(i * _L, _L)] = ones16
            return carry

        lax.fori_loop(0, _CHUNK // _L, fill_ones, 0)

        pltpu.sync_copy(stage_v, grid_sh.at[pl.ds(s * _SLAB, _SLAB)])
        plsc.subcore_barrier()

        half_lo = c * _HALF

        def make_point_body(b):
            def point_body(i, carry):
                o = i * _L
                x = xbufs[b][pl.ds(o, _L)]
                y = ybufs[b][pl.ds(o, _L)]
                z = zbufs[b][pl.ds(o, _L)]
                m = (
                    (x > _TLO[0]) & (x < _THI[0])
                    & (y > _TLO[1]) & (y < _THI[1])
                    & (z > _TLO[2]) & (z < _THI[2])
                )
                xq = ((x - _LO[0]) * float(_SPATIAL[0]) / _RANGE[0]).astype(jnp.int32)
                yq = ((y - _LO[1]) * float(_SPATIAL[1]) / _RANGE[1]).astype(jnp.int32)
                zq = ((z - _LO[2]) * float(_SPATIAL[2]) / _RANGE[2]).astype(jnp.int32)
                key = (xq * _SPATIAL[1] + yq) * _SPATIAL[2] + zq
                valid = m & (lax.shift_right_logical(key, 17) == c)
                kbufs[b][pl.ds(o, _L)] = jnp.where(valid, key - half_lo, -1)
                return carry
            return point_body

        scats = {}
        for k in range(_NCHUNKS):
            b = k % 2
            for cp in loads.pop(k):
                cp.wait()
            if k >= 2:
                scats.pop(k - 2).wait()
            iters = (_CHUNK if k < _NFULL else _REM) // _L
            lax.fori_loop(0, iters, make_point_body(b), 0)
            scats[k] = pltpu.async_copy(
                ones_v,
                grid_sh.at[plsc.Indices(kbufs[b], ignored_value=-1)],
                ssems[b])
            if k + 2 < _NCHUNKS:
                loads[k + 2] = issue_load(k + 2)

        for k in sorted(scats):
            scats.pop(k).wait()

        plsc.subcore_barrier()

        pltpu.sync_copy(grid_sh.at[pl.ds(s * _SLAB, _SLAB)], stage_v)
        out_base = pl.multiple_of(c * _HALF + s * _SLAB, 8)
        pltpu.sync_copy(stage_v, out_hbm.at[pl.ds(out_base, _SLAB)])

    return vox


_VOX = _make_kernel()


def kernel(pc, batch_idx):
    del batch_idx
    flat = _VOX(pc[:, 0], pc[:, 1], pc[:, 2])
    return flat.reshape(_SPATIAL)


# speedup vs baseline: 77.4918x; 1.2308x over previous
import functools

import numpy as np
import jax
import jax.numpy as jnp
from jax import lax
from jax.experimental import pallas as pl
from jax.experimental.pallas import tpu as pltpu
from jax.experimental.pallas import tpu_sc as plsc

_SPATIAL = (128, 128, 16)
_LO = np.array([0.0, -25.6, -2.0], dtype=np.float32)
_HI = np.array([51.2, 25.6, 4.4], dtype=np.float32)
_RANGE = _HI - _LO
_TLO = (_LO.astype(np.float64) + 1e-4).astype(np.float32)
_THI = (_HI.astype(np.float64) - 1e-4).astype(np.float32)

_N = 529408
_NC, _NS, _L = 2, 16, 16
_PT = _N // _NS
_CHUNK = 4096
_NFULL = _PT // _CHUNK
_REM = _PT - _NFULL * _CHUNK
_NCHUNKS = _NFULL + 1
_CELLS = _SPATIAL[0] * _SPATIAL[1] * _SPATIAL[2]
_HALF = _CELLS // 2
_SLAB = _HALF // _NS
_NKB = 3


def _make_kernel():
    mesh = plsc.VectorSubcoreMesh(core_axis_name="c", subcore_axis_name="s")

    @functools.partial(
        pl.kernel,
        out_type=jax.ShapeDtypeStruct((_CELLS,), jnp.float32),
        mesh=mesh,
        compiler_params=pltpu.CompilerParams(needs_layout_passes=False),
        scratch_types=[
            pltpu.VMEM((_CHUNK,), jnp.float32),
            pltpu.VMEM((_CHUNK,), jnp.float32),
            pltpu.VMEM((_CHUNK,), jnp.float32),
            pltpu.VMEM((_CHUNK,), jnp.float32),
            pltpu.VMEM((_CHUNK,), jnp.float32),
            pltpu.VMEM((_CHUNK,), jnp.float32),
            pltpu.VMEM((_CHUNK,), jnp.int32),
            pltpu.VMEM((_CHUNK,), jnp.int32),
            pltpu.VMEM((_CHUNK,), jnp.int32),
            pltpu.VMEM((_CHUNK,), jnp.float32),
            pltpu.VMEM((_SLAB,), jnp.float32),
            pltpu.VMEM_SHARED((_HALF,), jnp.float32),
            pltpu.SemaphoreType.DMA,
            pltpu.SemaphoreType.DMA,
            pltpu.SemaphoreType.DMA,
            pltpu.SemaphoreType.DMA,
            pltpu.SemaphoreType.DMA,
        ],
    )
    def vox(x_hbm, y_hbm, z_hbm, out_hbm,
            xv0, xv1, yv0, yv1, zv0, zv1, kb0, kb1, kb2,
            ones_v, stage_v, grid_sh,
            lsem0, lsem1, ssem0, ssem1, ssem2):
        c = lax.axis_index("c")
        s = lax.axis_index("s")
        zeros16 = jnp.zeros((_L,), jnp.float32)
        ones16 = jnp.ones((_L,), jnp.float32)

        kbufs = (kb0, kb1, kb2)
        ssems = (ssem0, ssem1, ssem2)
        lsems = (lsem0, lsem1)
        xbufs = (xv0, xv1)
        ybufs = (yv0, yv1)
        zbufs = (zv0, zv1)
        pts_base = s * _PT

        def issue_load(k):
            b = k % 2
            cnt = _CHUNK if k < _NFULL else _REM
            base = pl.multiple_of(pts_base + k * _CHUNK, 8)
            cps = []
            for hbm, buf in ((x_hbm, xbufs[b]), (y_hbm, ybufs[b]),
                             (z_hbm, zbufs[b])):
                dst = buf if cnt == _CHUNK else buf.at[pl.ds(0, cnt)]
                cps.append(pltpu.async_copy(
                    hbm.at[pl.ds(base, cnt)], dst, lsems[b]))
            return cps

        loads = {0: issue_load(0), 1: issue_load(1)}

        def fill_stage(i, carry):
            stage_v[pl.ds(i * _L, _L)] = zeros16
            return carry

        lax.fori_loop(0, _SLAB // _L, fill_stage, 0)

        def fill_ones(i, carry):
            ones_v[pl.ds(i * _L, _L)] = ones16
            return carry

        lax.fori_loop(0, _CHUNK // _L, fill_ones, 0)

        pltpu.sync_copy(stage_v, grid_sh.at[pl.ds(s * _SLAB, _SLAB)])
        plsc.subcore_barrier()

        xhalf = c * (_SPATIAL[0] // 2)

        def make_point_body(lb, kb):
            def point_body(i, carry):
                for j in range(2):
                    o = i * (2 * _L) + j * _L
                    x = xbufs[lb][pl.ds(o, _L)]
                    y = ybufs[lb][pl.ds(o, _L)]
                    z = zbufs[lb][pl.ds(o, _L)]
                    m = (
                        (x > _TLO[0]) & (x < _THI[0])
                        & (y > _TLO[1]) & (y < _THI[1])
                        & (z > _TLO[2]) & (z < _THI[2])
                    )
                    xq = ((x - _LO[0]) * float(_SPATIAL[0]) / _RANGE[0]).astype(jnp.int32)
                    yq = ((y - _LO[1]) * float(_SPATIAL[1]) / _RANGE[1]).astype(jnp.int32)
                    zq = ((z - _LO[2]) * float(_SPATIAL[2]) / _RANGE[2]).astype(jnp.int32)
                    key = ((xq - xhalf) * _SPATIAL[2] + zq) * _SPATIAL[1] + yq
                    valid = m & (lax.shift_right_arithmetic(xq, 6) == c)
                    kbufs[kb][pl.ds(o, _L)] = jnp.where(valid, key, -1)
                return carry
            return point_body

        scats = {}
        for k in range(_NCHUNKS):
            lb = k % 2
            kb = k % _NKB
            for cp in loads.pop(k):
                cp.wait()
            if k >= _NKB:
                scats.pop(k - _NKB).wait()
            iters = (_CHUNK if k < _NFULL else _REM) // (2 * _L)
            lax.fori_loop(0, iters, make_point_body(lb, kb), 0)
            scats[k] = pltpu.async_copy(
                ones_v,
                grid_sh.at[plsc.Indices(kbufs[kb], ignored_value=-1)],
                ssems[kb])
            if k + 2 < _NCHUNKS:
                loads[k + 2] = issue_load(k + 2)

        for k in sorted(scats):
            scats.pop(k).wait()

        plsc.subcore_barrier()

        pltpu.sync_copy(grid_sh.at[pl.ds(s * _SLAB, _SLAB)], stage_v)
        out_base = pl.multiple_of(c * _HALF + s * _SLAB, 8)
        pltpu.sync_copy(stage_v, out_hbm.at[pl.ds(out_base, _SLAB)])

    return vox


_VOX = _make_kernel()


def kernel(pc, batch_idx):
    del batch_idx
    flat = _VOX(pc[:, 0], pc[:, 1], pc[:, 2])
    x, y, z = _SPATIAL
    return flat.reshape(x, z, y).transpose(0, 2, 1)


# speedup vs baseline: 79.2003x; 1.0220x over previous
import functools

import numpy as np
import jax
import jax.numpy as jnp
from jax import lax
from jax.experimental import pallas as pl
from jax.experimental.pallas import tpu as pltpu
from jax.experimental.pallas import tpu_sc as plsc

_SPATIAL = (128, 128, 16)
_LO = np.array([0.0, -25.6, -2.0], dtype=np.float32)
_HI = np.array([51.2, 25.6, 4.4], dtype=np.float32)
_RANGE = _HI - _LO
_TLO = (_LO.astype(np.float64) + 1e-4).astype(np.float32)
_THI = (_HI.astype(np.float64) - 1e-4).astype(np.float32)

_N = 529408
_NC, _NS, _L = 2, 16, 16
_PT = _N // _NS
_CHUNK = 4096
_NFULL = _PT // _CHUNK
_REM = _PT - _NFULL * _CHUNK
_NCHUNKS = _NFULL + 1
_CELLS = _SPATIAL[0] * _SPATIAL[1] * _SPATIAL[2]
_HALF = _CELLS // 2
_SLAB = _HALF // _NS
_NKB = 3


def _make_kernel():
    mesh = plsc.VectorSubcoreMesh(core_axis_name="c", subcore_axis_name="s")

    @functools.partial(
        pl.kernel,
        out_type=jax.ShapeDtypeStruct((_CELLS,), jnp.float32),
        mesh=mesh,
        compiler_params=pltpu.CompilerParams(needs_layout_passes=False),
        scratch_types=[
            pltpu.VMEM((_CHUNK,), jnp.float32),
            pltpu.VMEM((_CHUNK,), jnp.float32),
            pltpu.VMEM((_CHUNK,), jnp.float32),
            pltpu.VMEM((_CHUNK,), jnp.float32),
            pltpu.VMEM((_CHUNK,), jnp.float32),
            pltpu.VMEM((_CHUNK,), jnp.float32),
            pltpu.VMEM((_CHUNK,), jnp.int32),
            pltpu.VMEM((_CHUNK,), jnp.int32),
            pltpu.VMEM((_CHUNK,), jnp.int32),
            pltpu.VMEM((_CHUNK,), jnp.float32),
            pltpu.VMEM((_SLAB,), jnp.float32),
            pltpu.VMEM_SHARED((_HALF,), jnp.float32),
            pltpu.SemaphoreType.DMA,
            pltpu.SemaphoreType.DMA,
            pltpu.SemaphoreType.DMA,
            pltpu.SemaphoreType.DMA,
            pltpu.SemaphoreType.DMA,
        ],
    )
    def vox(x_hbm, y_hbm, z_hbm, out_hbm,
            xv0, xv1, yv0, yv1, zv0, zv1, kb0, kb1, kb2,
            ones_v, stage_v, grid_sh,
            lsem0, lsem1, ssem0, ssem1, ssem2):
        c = lax.axis_index("c")
        s = lax.axis_index("s")
        zeros16 = jnp.zeros((_L,), jnp.float32)
        ones16 = jnp.ones((_L,), jnp.float32)

        kbufs = (kb0, kb1, kb2)
        ssems = (ssem0, ssem1, ssem2)
        lsems = (lsem0, lsem1)
        xbufs = (xv0, xv1)
        ybufs = (yv0, yv1)
        zbufs = (zv0, zv1)
        pts_base = s * _PT

        def issue_load(k):
            b = k % 2
            cnt = _CHUNK if k < _NFULL else _REM
            base = pl.multiple_of(pts_base + k * _CHUNK, 8)
            cps = []
            for hbm, buf in ((x_hbm, xbufs[b]), (y_hbm, ybufs[b]),
                             (z_hbm, zbufs[b])):
                dst = buf if cnt == _CHUNK else buf.at[pl.ds(0, cnt)]
                cps.append(pltpu.async_copy(
                    hbm.at[pl.ds(base, cnt)], dst, lsems[b]))
            return cps

        loads = {0: issue_load(0), 1: issue_load(1)}

        def fill_stage(i, carry):
            stage_v[pl.ds(i * _L, _L)] = zeros16
            return carry

        lax.fori_loop(0, _SLAB // _L, fill_stage, 0)

        def fill_ones(i, carry):
            ones_v[pl.ds(i * _L, _L)] = ones16
            return carry

        lax.fori_loop(0, _CHUNK // _L, fill_ones, 0)

        pltpu.sync_copy(stage_v, grid_sh.at[pl.ds(s * _SLAB, _SLAB)])
        plsc.subcore_barrier()

        xhalf = c * (_SPATIAL[0] // 2)

        def emit_chunk(lb, kb, iters):
            @plsc.parallel_loop(0, iters, 1, unroll=4)
            def _chunk(i):
                o = i * _L
                x = xbufs[lb][pl.ds(o, _L)]
                y = ybufs[lb][pl.ds(o, _L)]
                z = zbufs[lb][pl.ds(o, _L)]
                m = (
                    (x > _TLO[0]) & (x < _THI[0])
                    & (y > _TLO[1]) & (y < _THI[1])
                    & (z > _TLO[2]) & (z < _THI[2])
                )
                xq = ((x - _LO[0]) * float(_SPATIAL[0]) / _RANGE[0]).astype(jnp.int32)
                yq = ((y - _LO[1]) * float(_SPATIAL[1]) / _RANGE[1]).astype(jnp.int32)
                zq = ((z - _LO[2]) * float(_SPATIAL[2]) / _RANGE[2]).astype(jnp.int32)
                key = ((xq - xhalf) * _SPATIAL[2] + zq) * _SPATIAL[1] + yq
                valid = m & (lax.shift_right_arithmetic(xq, 6) == c)
                kbufs[kb][pl.ds(o, _L)] = jnp.where(valid, key, -1)

        scats = {}
        for k in range(_NCHUNKS):
            lb = k % 2
            kb = k % _NKB
            for cp in loads.pop(k):
                cp.wait()
            if k >= _NKB:
                scats.pop(k - _NKB).wait()
            emit_chunk(lb, kb, (_CHUNK if k < _NFULL else _REM) // _L)
            scats[k] = pltpu.async_copy(
                ones_v,
                grid_sh.at[plsc.Indices(kbufs[kb], ignored_value=-1)],
                ssems[kb])
            if k + 2 < _NCHUNKS:
                loads[k + 2] = issue_load(k + 2)

        for k in sorted(scats):
            scats.pop(k).wait()

        plsc.subcore_barrier()

        pltpu.sync_copy(grid_sh.at[pl.ds(s * _SLAB, _SLAB)], stage_v)
        out_base = pl.multiple_of(c * _HALF + s * _SLAB, 8)
        pltpu.sync_copy(stage_v, out_hbm.at[pl.ds(out_base, _SLAB)])

    return vox


_VOX = _make_kernel()


def kernel(pc, batch_idx):
    del batch_idx
    flat = _VOX(pc[:, 0], pc[:, 1], pc[:, 2])
    x, y, z = _SPATIAL
    return flat.reshape(x, z, y).transpose(0, 2, 1)
